# baseline (speedup 1.0000x reference)
import functools

import jax
import jax.numpy as jnp
from jax import lax
from jax.experimental import pallas as pl
from jax.experimental.pallas import tpu as pltpu
from jax.experimental.pallas import tpu_sc as plsc

N = 10000
E = 320000
IN_C = 128
HID_C = 64
OUT_C = 128

_LANES = 128
_EROWS_PER_TILE = 80
_EROWS = 32 * _EROWS_PER_TILE
_E_PAD = _EROWS * _LANES
_IDX_BLK = 8
_N_ACC = 10112
_RPT = _N_ACC // 16

_MESH = plsc.VectorSubcoreMesh(core_axis_name="c", subcore_axis_name="s")
_SC_PARAMS = pltpu.CompilerParams(use_tc_tiling_on_sc=False)


@functools.partial(
    pl.kernel,
    out_type=jax.ShapeDtypeStruct((2, _N_ACC, 16), jnp.float32),
    mesh=_MESH,
    compiler_params=_SC_PARAMS,
    scratch_types=[
        pltpu.VMEM((_EROWS_PER_TILE, _LANES), jnp.int32),
        pltpu.VMEM((_LANES, 16), jnp.float32),
        pltpu.VMEM((_RPT, 16), jnp.float32),
        pltpu.VMEM_SHARED((_N_ACC, 16), jnp.float32),
        pltpu.SemaphoreType.DMA,
        pltpu.SemaphoreType.DMA,
    ],
)
def _sc_degree(dstm_hbm, out_hbm, idv, ones_v, zero_v, acc, sem_i, sem_s):
    c = lax.axis_index("c")
    s = lax.axis_index("s")
    wid = c * 16 + s
    base = wid * _EROWS_PER_TILE

    cpi = pltpu.async_copy(dstm_hbm.at[pl.ds(base, _EROWS_PER_TILE)], idv,
                           sem_i)

    @pl.loop(0, _RPT)
    def _(i):
        zero_v[i, :] = jnp.zeros((16,), jnp.float32)

    @pl.loop(0, _LANES)
    def _(j):
        ones_v[j, :] = jnp.ones((16,), jnp.float32)

    r0 = s * _RPT
    pltpu.sync_copy(zero_v, acc.at[pl.ds(r0, _RPT)])
    cpi.wait()
    plsc.subcore_barrier()

    @pl.loop(0, _EROWS_PER_TILE // _IDX_BLK)
    def _(i):
        for j in range(_IDX_BLK):
            pltpu.async_copy(ones_v, acc.at[idv.at[i * _IDX_BLK + j]], sem_s,
                             add=True)

        @pl.when(i > 0)
        def _():
            for j in range(_IDX_BLK):
                pltpu.make_async_copy(ones_v, acc.at[idv.at[j]], sem_s).wait()

    for j in range(_IDX_BLK):
        pltpu.make_async_copy(ones_v, acc.at[idv.at[j]], sem_s).wait()

    plsc.subcore_barrier()
    pltpu.sync_copy(acc.at[pl.ds(r0, _RPT)], out_hbm.at[c, pl.ds(r0, _RPT)])


_RPB = 4
_EROWS_C0 = 88
_EROWS_C1 = 72
_EROWS_MAX = 88


@functools.partial(
    pl.kernel,
    out_type=jax.ShapeDtypeStruct((2, _N_ACC, HID_C), jnp.float32),
    mesh=_MESH,
    compiler_params=_SC_PARAMS,
    scratch_types=[
        pltpu.VMEM((_EROWS_MAX, _LANES), jnp.int32),
        pltpu.VMEM((_EROWS_MAX, _LANES), jnp.int32),
        pltpu.VMEM((2, _RPB * _LANES, HID_C), jnp.float32),
        pltpu.VMEM_SHARED((_N_ACC, HID_C), jnp.float32),
        pltpu.SemaphoreType.DMA,
        pltpu.SemaphoreType.DMA,
        pltpu.SemaphoreType.DMA,
    ],
)
def _sc_propagate(h_hbm, srcm_hbm, dstm_hbm, out_hbm, isv, idv, rows_v, acc,
                  sem_g, sem_s0, sem_s1):
    c = lax.axis_index("c")
    s = lax.axis_index("s")
    sem_s = (sem_s0, sem_s1)

    r0 = s * _RPT
    base = lax.select(c == 0, 16 * _EROWS_C1 + s * _EROWS_C0,
                      s * _EROWS_C1)
    nblk2 = lax.select(c == 0, _EROWS_C0 // (2 * _RPB),
                       _EROWS_C1 // (2 * _RPB))
    cp0 = pltpu.async_copy(h_hbm.at[pl.ds(r0, _RPT)],
                           acc.at[pl.ds(r0, _RPT)], sem_g)
    cp1 = pltpu.async_copy(srcm_hbm.at[pl.ds(base, _EROWS_MAX)], isv, sem_g)
    cp2 = pltpu.async_copy(dstm_hbm.at[pl.ds(base, _EROWS_MAX)], idv, sem_g)
    cp0.wait()
    cp1.wait()
    cp2.wait()
    plsc.subcore_barrier()

    @pl.loop(0, 11)
    def _(g):
      @pl.when(g < nblk2)
      def _():
        for buf in range(2):
            b = 2 * g + buf

            @pl.when(g > 0)
            def _():
                for j in range(_RPB):
                    pltpu.make_async_copy(
                        rows_v.at[buf, pl.ds(j * _LANES, _LANES)],
                        acc.at[idv.at[b * _RPB + j]], sem_s[buf]).wait()

            for j in range(_RPB):
                pltpu.async_copy(h_hbm.at[isv.at[b * _RPB + j]],
                                 rows_v.at[buf, pl.ds(j * _LANES, _LANES)],
                                 sem_g)
            for j in range(_RPB):
                pltpu.make_async_copy(h_hbm.at[isv.at[b * _RPB + j]],
                                      rows_v.at[buf, pl.ds(j * _LANES, _LANES)],
                                      sem_g).wait()
            for j in range(_RPB):
                pltpu.async_copy(rows_v.at[buf, pl.ds(j * _LANES, _LANES)],
                                 acc.at[idv.at[b * _RPB + j]], sem_s[buf],
                                 add=True)

    for buf in range(2):
        b = 2 * nblk2 - 2 + buf
        for j in range(_RPB):
            pltpu.make_async_copy(rows_v.at[buf, pl.ds(j * _LANES, _LANES)],
                                  acc.at[idv.at[b * _RPB + j]],
                                  sem_s[buf]).wait()

    plsc.subcore_barrier()
    pltpu.sync_copy(acc.at[pl.ds(r0, _RPT)], out_hbm.at[c, pl.ds(r0, _RPT)])


def _dinv_full(d_ref):
    deg = d_ref[0, :, 0:1] + d_ref[1, :, 0:1] + 1.0
    return lax.rsqrt(jnp.maximum(deg, 1e-12))


def _tc_hs1(x, w1, degp):
    def body(x_ref, w_ref, d_ref, o_ref):
        dinv = _dinv_full(d_ref)
        h1 = jnp.dot(x_ref[...], w_ref[...],
                     preferred_element_type=jnp.float32)
        o_ref[0:N, :] = h1 * dinv[0:N, :]
        o_ref[N:_N_ACC, :] = jnp.zeros((_N_ACC - N, HID_C), jnp.float32)

    return pl.pallas_call(
        body, out_shape=jax.ShapeDtypeStruct((_N_ACC, HID_C), jnp.float32))(
            x, w1, degp)


def _tc_mid(p, hs1, degp, b1):
    def body(p_ref, hs_ref, d_ref, b_ref, o_ref):
        dinv = _dinv_full(d_ref)
        seg = p_ref[0] + p_ref[1] - hs_ref[...]
        hid = jnp.maximum(seg * dinv + b_ref[...], 0.0)
        o_ref[0:N, :] = (hid * dinv)[0:N, :]
        o_ref[N:_N_ACC, :] = jnp.zeros((_N_ACC - N, HID_C), jnp.float32)

    return pl.pallas_call(
        body, out_shape=jax.ShapeDtypeStruct((_N_ACC, HID_C), jnp.float32))(
            p, hs1, degp, b1)


def _tc_out(p, hs2, degp, w2, b2):
    def body(p_ref, hs_ref, d_ref, w_ref, b_ref, o_ref):
        deg = d_ref[0, 0:N, 0:1] + d_ref[1, 0:N, 0:1] + 1.0
        dinv = lax.rsqrt(jnp.maximum(deg, 1e-12))
        q = (p_ref[0, 0:N, :] + p_ref[1, 0:N, :] - hs_ref[0:N, :]) * dinv
        z = jnp.dot(q, w_ref[...], preferred_element_type=jnp.float32)
        z = z + b_ref[...]
        zs = z - jnp.max(z, axis=1, keepdims=True)
        lse = jnp.log(jnp.sum(jnp.exp(zs), axis=1, keepdims=True))
        o_ref[...] = zs - lse

    return pl.pallas_call(
        body, out_shape=jax.ShapeDtypeStruct((N, OUT_C), jnp.float32))(
            p, hs2, degp, w2, b2)


def kernel(x, edge_index, W1, b1, W2, b2):
    src = edge_index[0]
    dst = edge_index[1]
    fill = jnp.arange(_E_PAD - E, dtype=jnp.int32) % 16
    srcm = jnp.concatenate([src, fill]).reshape(_EROWS, _LANES)
    dstm = jnp.concatenate([dst, fill + N]).reshape(_EROWS, _LANES)

    degp = _sc_degree(dstm)
    hs1 = _tc_hs1(x, W1, degp)
    p1 = _sc_propagate(hs1, srcm, dstm)
    hs2 = _tc_mid(p1, hs1, degp, b1.reshape(1, HID_C))
    p2 = _sc_propagate(hs2, srcm, dstm)
    return _tc_out(p2, hs2, degp, W2, b2.reshape(1, OUT_C))


# speedup vs baseline: 1.0501x; 1.0501x over previous
import functools

import jax
import jax.numpy as jnp
from jax import lax
from jax.experimental import pallas as pl
from jax.experimental.pallas import tpu as pltpu
from jax.experimental.pallas import tpu_sc as plsc

N = 10000
E = 320000
IN_C = 128
HID_C = 64
OUT_C = 128

_LANES = 128
_EROWS_PER_TILE = 80
_EROWS = 32 * _EROWS_PER_TILE
_E_PAD = _EROWS * _LANES
_IDX_BLK = 8
_N_ACC = 10112
_RPT = _N_ACC // 16

_MESH = plsc.VectorSubcoreMesh(core_axis_name="c", subcore_axis_name="s")
_SC_PARAMS = pltpu.CompilerParams(use_tc_tiling_on_sc=False)


@functools.partial(
    pl.kernel,
    out_type=jax.ShapeDtypeStruct((2, _N_ACC, 16), jnp.float32),
    mesh=_MESH,
    compiler_params=_SC_PARAMS,
    scratch_types=[
        pltpu.VMEM((_EROWS_PER_TILE, _LANES), jnp.int32),
        pltpu.VMEM((_LANES, 16), jnp.float32),
        pltpu.VMEM((_RPT, 16), jnp.float32),
        pltpu.VMEM_SHARED((_N_ACC, 16), jnp.float32),
        pltpu.SemaphoreType.DMA,
        pltpu.SemaphoreType.DMA,
    ],
)
def _sc_degree(dstm_hbm, out_hbm, idv, ones_v, zero_v, acc, sem_i, sem_s):
    c = lax.axis_index("c")
    s = lax.axis_index("s")
    wid = c * 16 + s
    base = wid * _EROWS_PER_TILE

    cpi = pltpu.async_copy(dstm_hbm.at[pl.ds(base, _EROWS_PER_TILE)], idv,
                           sem_i)

    @pl.loop(0, _RPT)
    def _(i):
        zero_v[i, :] = jnp.zeros((16,), jnp.float32)

    @pl.loop(0, _LANES)
    def _(j):
        ones_v[j, :] = jnp.ones((16,), jnp.float32)

    r0 = s * _RPT
    pltpu.sync_copy(zero_v, acc.at[pl.ds(r0, _RPT)])
    cpi.wait()
    plsc.subcore_barrier()

    @pl.loop(0, _EROWS_PER_TILE // _IDX_BLK)
    def _(i):
        for j in range(_IDX_BLK):
            pltpu.async_copy(ones_v, acc.at[idv.at[i * _IDX_BLK + j]], sem_s,
                             add=True)

        @pl.when(i > 0)
        def _():
            for j in range(_IDX_BLK):
                pltpu.make_async_copy(ones_v, acc.at[idv.at[j]], sem_s).wait()

    for j in range(_IDX_BLK):
        pltpu.make_async_copy(ones_v, acc.at[idv.at[j]], sem_s).wait()

    plsc.subcore_barrier()
    pltpu.sync_copy(acc.at[pl.ds(r0, _RPT)], out_hbm.at[c, pl.ds(r0, _RPT)])


_RPB = 4
_NBLK = _EROWS_PER_TILE // _RPB


@functools.partial(
    pl.kernel,
    out_type=jax.ShapeDtypeStruct((2, _N_ACC, HID_C), jnp.float32),
    mesh=_MESH,
    compiler_params=_SC_PARAMS,
    scratch_types=[
        pltpu.VMEM((_EROWS_PER_TILE, _LANES), jnp.int32),
        pltpu.VMEM((_EROWS_PER_TILE, _LANES), jnp.int32),
        pltpu.VMEM((2, _RPB * _LANES, HID_C), jnp.float32),
        pltpu.VMEM_SHARED((_N_ACC, HID_C), jnp.float32),
        pltpu.SemaphoreType.DMA,
        pltpu.SemaphoreType.DMA,
        pltpu.SemaphoreType.DMA,
    ],
)
def _sc_propagate(h_hbm, srcm_hbm, dstm_hbm, out_hbm, isv, idv, rows_v, acc,
                  sem_g, sem_s0, sem_s1):
    c = lax.axis_index("c")
    s = lax.axis_index("s")
    wid = c * 16 + s
    sem_s = (sem_s0, sem_s1)

    r0 = s * _RPT
    base = wid * _EROWS_PER_TILE
    cp0 = pltpu.async_copy(h_hbm.at[pl.ds(r0, _RPT)],
                           acc.at[pl.ds(r0, _RPT)], sem_g)
    cp1 = pltpu.async_copy(srcm_hbm.at[pl.ds(base, _EROWS_PER_TILE)], isv,
                           sem_g)
    cp2 = pltpu.async_copy(dstm_hbm.at[pl.ds(base, _EROWS_PER_TILE)], idv,
                           sem_g)
    cp0.wait()
    cp1.wait()
    cp2.wait()
    plsc.subcore_barrier()

    @pl.loop(0, _NBLK // 2)
    def _(g):
        for buf in range(2):
            b = 2 * g + buf

            @pl.when(g > 0)
            def _():
                for j in range(_RPB):
                    pltpu.make_async_copy(
                        rows_v.at[buf, pl.ds(j * _LANES, _LANES)],
                        acc.at[idv.at[b * _RPB + j]], sem_s[buf]).wait()

            for j in range(_RPB):
                pltpu.async_copy(h_hbm.at[isv.at[b * _RPB + j]],
                                 rows_v.at[buf, pl.ds(j * _LANES, _LANES)],
                                 sem_g)
            for j in range(_RPB):
                pltpu.make_async_copy(h_hbm.at[isv.at[b * _RPB + j]],
                                      rows_v.at[buf, pl.ds(j * _LANES, _LANES)],
                                      sem_g).wait()
            for j in range(_RPB):
                pltpu.async_copy(rows_v.at[buf, pl.ds(j * _LANES, _LANES)],
                                 acc.at[idv.at[b * _RPB + j]], sem_s[buf],
                                 add=True)

    for buf in range(2):
        b = _NBLK - 2 + buf
        for j in range(_RPB):
            pltpu.make_async_copy(rows_v.at[buf, pl.ds(j * _LANES, _LANES)],
                                  acc.at[idv.at[b * _RPB + j]],
                                  sem_s[buf]).wait()

    plsc.subcore_barrier()
    pltpu.sync_copy(acc.at[pl.ds(r0, _RPT)], out_hbm.at[c, pl.ds(r0, _RPT)])


def _dinv_full(d_ref):
    deg = d_ref[0, :, 0:1] + d_ref[1, :, 0:1] + 1.0
    return lax.rsqrt(jnp.maximum(deg, 1e-12))


def _tc_hs1(x, w1, degp):
    def body(x_ref, w_ref, d_ref, o_ref):
        dinv = _dinv_full(d_ref)
        h1 = jnp.dot(x_ref[...], w_ref[...],
                     preferred_element_type=jnp.float32)
        o_ref[0:N, :] = h1 * dinv[0:N, :]
        o_ref[N:_N_ACC, :] = jnp.zeros((_N_ACC - N, HID_C), jnp.float32)

    return pl.pallas_call(
        body, out_shape=jax.ShapeDtypeStruct((_N_ACC, HID_C), jnp.float32))(
            x, w1, degp)


def _tc_mid(p, hs1, degp, b1):
    def body(p_ref, hs_ref, d_ref, b_ref, o_ref):
        dinv = _dinv_full(d_ref)
        seg = p_ref[0] + p_ref[1] - hs_ref[...]
        hid = jnp.maximum(seg * dinv + b_ref[...], 0.0)
        o_ref[0:N, :] = (hid * dinv)[0:N, :]
        o_ref[N:_N_ACC, :] = jnp.zeros((_N_ACC - N, HID_C), jnp.float32)

    return pl.pallas_call(
        body, out_shape=jax.ShapeDtypeStruct((_N_ACC, HID_C), jnp.float32))(
            p, hs1, degp, b1)


def _tc_out(p, hs2, degp, w2, b2):
    def body(p_ref, hs_ref, d_ref, w_ref, b_ref, o_ref):
        deg = d_ref[0, 0:N, 0:1] + d_ref[1, 0:N, 0:1] + 1.0
        dinv = lax.rsqrt(jnp.maximum(deg, 1e-12))
        q = (p_ref[0, 0:N, :] + p_ref[1, 0:N, :] - hs_ref[0:N, :]) * dinv
        z = jnp.dot(q, w_ref[...], preferred_element_type=jnp.float32)
        z = z + b_ref[...]
        zs = z - jnp.max(z, axis=1, keepdims=True)
        lse = jnp.log(jnp.sum(jnp.exp(zs), axis=1, keepdims=True))
        o_ref[...] = zs - lse

    return pl.pallas_call(
        body, out_shape=jax.ShapeDtypeStruct((N, OUT_C), jnp.float32))(
            p, hs2, degp, w2, b2)


def kernel(x, edge_index, W1, b1, W2, b2):
    src = edge_index[0]
    dst = edge_index[1]
    fill = jnp.arange(_E_PAD - E, dtype=jnp.int32) % 16
    srcm = jnp.concatenate([src, fill]).reshape(_EROWS, _LANES)
    dstm = jnp.concatenate([dst, fill + N]).reshape(_EROWS, _LANES)

    degp = _sc_degree(dstm)
    hs1 = _tc_hs1(x, W1, degp)
    p1 = _sc_propagate(hs1, srcm, dstm)
    hs2 = _tc_mid(p1, hs1, degp, b1.reshape(1, HID_C))
    p2 = _sc_propagate(hs2, srcm, dstm)
    return _tc_out(p2, hs2, degp, W2, b2.reshape(1, OUT_C))
